# baseline (speedup 1.0000x reference)
import functools

import jax
import jax.numpy as jnp
from jax import lax
from jax.experimental import pallas as pl
from jax.experimental.pallas import tpu as pltpu
from jax.experimental.pallas import tpu_sc as plsc

N = 1000000
D = 64
B = 16384

NC = 2
NS = 16
NW = NC * NS
TOTAL = 2 * B
IDX_PER_W = TOTAL // NW
ROWS_PER_W = IDX_PER_W // 2
PIPE = 64

TW = 32768
TW2 = TW // 2
TGRID = -(-N // TW)
LINES = TGRID * TW2


def _transpose_body(xt_ref, eye_ref, out_ref):
    xt = xt_ref[...]
    eye = eye_ref[...]
    dn = (((0,), (0,)), ((), ()))
    out_ref[:, 0:D] = lax.dot_general(
        xt[:, 0:TW2], eye, dn, preferred_element_type=jnp.float32
    )
    out_ref[:, D : 2 * D] = lax.dot_general(
        xt[:, TW2:TW], eye, dn, preferred_element_type=jnp.float32
    )


def _tc_transpose(embT):
    eye = jnp.eye(D, dtype=jnp.float32)
    return pl.pallas_call(
        _transpose_body,
        grid=(TGRID,),
        in_specs=[
            pl.BlockSpec((D, TW), lambda i: (0, i)),
            pl.BlockSpec((D, D), lambda i: (0, 0)),
        ],
        out_specs=pl.BlockSpec((TW2, 2 * D), lambda i: (i, 0)),
        out_shape=jax.ShapeDtypeStruct((LINES, 2 * D), jnp.float32),
    )(embT, eye)


GS = 16
NG = IDX_PER_W // GS
RING = 8


def _sc_gather(emb2, idx):
    mesh = plsc.VectorSubcoreMesh(core_axis_name="c", subcore_axis_name="s")

    @functools.partial(
        pl.kernel,
        out_type=jax.ShapeDtypeStruct((B, 2 * D), jnp.float32),
        mesh=mesh,
        scratch_types=[
            pltpu.VMEM((IDX_PER_W,), jnp.int32),
            pltpu.VMEM((RING, GS, 2 * D), jnp.float32),
            pltpu.VMEM((ROWS_PER_W, 2 * D), jnp.float32),
            [pltpu.SemaphoreType.DMA] * RING,
        ],
    )
    def gather_kernel(emb_hbm, idx_hbm, out_hbm, idx_v, stage, rowbuf, sems):
        wid = lax.axis_index("s") * NC + lax.axis_index("c")
        base_idx = wid * IDX_PER_W
        base_row = wid * ROWS_PER_W
        pltpu.sync_copy(idx_hbm.at[pl.ds(base_idx, IDX_PER_W)], idx_v)

        def issue(g, s):
            k16 = idx_v[pl.ds(g * GS, GS)]
            for j in range(GS):
                k = k16[j]
                line = lax.bitwise_or(
                    lax.shift_left(lax.shift_right_logical(k, 15), 14),
                    lax.bitwise_and(k, TW2 - 1),
                )
                pltpu.make_async_copy(
                    emb_hbm.at[line], stage.at[s, j], sems[s]
                ).start()

        def drain_extract(g, s):
            for j in range(GS):
                pltpu.make_async_copy(
                    emb_hbm.at[0], stage.at[s, 0], sems[s]
                ).wait()
            k16 = idx_v[pl.ds(g * GS, GS)]
            for j in range(GS):
                off = lax.bitwise_and(
                    lax.shift_right_logical(k16[j], 14), 1
                ) * D
                orow = g * (GS // 2) + (j // 2)
                colh = (j % 2) * D
                for t in range(D // 16):
                    rowbuf[orow, pl.ds(colh + t * 16, 16)] = stage[
                        s, j, pl.ds(off + t * 16, 16)
                    ]

        for s in range(RING):
            issue(s, s)

        def body(q, _):
            for s in range(RING):
                g = q * RING + s
                drain_extract(g, s)

                @pl.when(g + RING < NG)
                def _():
                    issue(g + RING, s)

            return _

        lax.fori_loop(0, NG // RING, body, None)
        pltpu.sync_copy(rowbuf, out_hbm.at[pl.ds(base_row, ROWS_PER_W)])

    return gather_kernel(emb2, idx)


BLK = 2048
GRID = B // BLK


def _mlp_loss_body(x_ref, w1_ref, b1_ref, w2_ref, b2_ref, lab_ref, out_ref):
    x = x_ref[...]
    h = jnp.maximum(
        jnp.dot(x, w1_ref[...], preferred_element_type=jnp.float32)
        + b1_ref[...],
        0.0,
    )
    w2 = w2_ref[...]
    wd = w2[:, 0:1] - w2[:, 1:2]
    b2 = b2_ref[...]
    d = jnp.dot(h, wd, preferred_element_type=jnp.float32) + (
        b2[0, 0] - b2[0, 1]
    )
    sign = (2 * lab_ref[...] - 1).astype(jnp.float32)
    z = sign * d
    nll = jnp.maximum(z, 0.0) + jnp.log1p(jnp.exp(-jnp.abs(z)))
    partial = jnp.sum(nll) * (1.0 / B)

    @pl.when(pl.program_id(0) == 0)
    def _():
        out_ref[0, 0] = 0.0

    out_ref[0, 0] += partial


def _tc_mlp_loss(x, w1, b1, w2, b2, labels):
    return pl.pallas_call(
        _mlp_loss_body,
        grid=(GRID,),
        in_specs=[
            pl.BlockSpec((BLK, 2 * D), lambda i: (i, 0)),
            pl.BlockSpec((2 * D, D), lambda i: (0, 0)),
            pl.BlockSpec((1, D), lambda i: (0, 0)),
            pl.BlockSpec((D, 2), lambda i: (0, 0)),
            pl.BlockSpec((1, 2), lambda i: (0, 0)),
            pl.BlockSpec((BLK, 1), lambda i: (i, 0)),
        ],
        out_specs=pl.BlockSpec(
            (1, 1), lambda i: (0, 0), memory_space=pltpu.SMEM
        ),
        out_shape=jax.ShapeDtypeStruct((1, 1), jnp.float32),
    )(x, w1, b1, w2, b2, labels)


def kernel(pairs, labels, emb, W1, b1, W2, b2):
    idx = pairs.astype(jnp.int32).reshape(TOTAL)
    emb_rm = _tc_transpose(emb.T)
    x = _sc_gather(emb_rm, idx)
    loss = _tc_mlp_loss(
        x,
        W1,
        b1.reshape(1, D),
        W2,
        b2.reshape(1, 2),
        labels.astype(jnp.int32).reshape(B, 1),
    )
    return loss[0, 0]


# speedup vs baseline: 1.0432x; 1.0432x over previous
import functools

import jax
import jax.numpy as jnp
from jax import lax
from jax.experimental import pallas as pl
from jax.experimental.pallas import tpu as pltpu
from jax.experimental.pallas import tpu_sc as plsc

N = 1000000
D = 64
B = 16384

NC = 2
NS = 16
NW = NC * NS
TOTAL = 2 * B
IDX_PER_W = TOTAL // NW
ROWS_PER_W = IDX_PER_W // 2
PIPE = 64

TW = 32768
TW2 = TW // 2
TGRID = -(-N // TW)
LINES = TGRID * TW2


def _transpose_body(xt_ref, eye_ref, out_ref):
    xt = xt_ref[...]
    eye = eye_ref[...]
    dn = (((0,), (0,)), ((), ()))
    out_ref[:, 0:D] = lax.dot_general(
        xt[:, 0:TW2], eye, dn, preferred_element_type=jnp.float32
    )
    out_ref[:, D : 2 * D] = lax.dot_general(
        xt[:, TW2:TW], eye, dn, preferred_element_type=jnp.float32
    )


def _tc_transpose(embT):
    eye = jnp.eye(D, dtype=jnp.float32)
    return pl.pallas_call(
        _transpose_body,
        grid=(TGRID,),
        in_specs=[
            pl.BlockSpec((D, TW), lambda i: (0, i)),
            pl.BlockSpec((D, D), lambda i: (0, 0)),
        ],
        out_specs=pl.BlockSpec((TW2, 2 * D), lambda i: (i, 0)),
        out_shape=jax.ShapeDtypeStruct((LINES, 2 * D), jnp.float32),
    )(embT, eye)


GS = 16
NG = IDX_PER_W // GS
RING = 4


def _sc_gather(emb2, idx):
    mesh = plsc.VectorSubcoreMesh(core_axis_name="c", subcore_axis_name="s")

    @functools.partial(
        pl.kernel,
        out_type=jax.ShapeDtypeStruct((B, 2 * D), jnp.float32),
        mesh=mesh,
        scratch_types=[
            pltpu.VMEM((IDX_PER_W,), jnp.int32),
            pltpu.VMEM((RING, GS, 2 * D), jnp.float32),
            pltpu.VMEM((ROWS_PER_W, 2 * D), jnp.float32),
            [pltpu.SemaphoreType.DMA] * RING,
        ],
    )
    def gather_kernel(emb_hbm, idx_hbm, out_hbm, idx_v, stage, rowbuf, sems):
        wid = lax.axis_index("s") * NC + lax.axis_index("c")
        base_idx = wid * IDX_PER_W
        base_row = wid * ROWS_PER_W
        pltpu.sync_copy(idx_hbm.at[pl.ds(base_idx, IDX_PER_W)], idx_v)

        def issue(g, s):
            k16 = idx_v[pl.ds(g * GS, GS)]
            for j in range(GS):
                k = k16[j]
                line = lax.bitwise_or(
                    lax.shift_left(lax.shift_right_logical(k, 15), 14),
                    lax.bitwise_and(k, TW2 - 1),
                )
                pltpu.make_async_copy(
                    emb_hbm.at[line], stage.at[s, j], sems[s]
                ).start()

        def drain_extract(g, s):
            for j in range(GS):
                pltpu.make_async_copy(
                    emb_hbm.at[0], stage.at[s, 0], sems[s]
                ).wait()
            k16 = idx_v[pl.ds(g * GS, GS)]
            for j in range(GS):
                off = lax.bitwise_and(
                    lax.shift_right_logical(k16[j], 14), 1
                ) * D
                orow = g * (GS // 2) + (j // 2)
                colh = (j % 2) * D
                for t in range(D // 16):
                    rowbuf[orow, pl.ds(colh + t * 16, 16)] = stage[
                        s, j, pl.ds(off + t * 16, 16)
                    ]

        for s in range(RING):
            issue(s, s)

        def body(q, _):
            for s in range(RING):
                g = q * RING + s
                drain_extract(g, s)

                @pl.when(g + RING < NG)
                def _():
                    issue(g + RING, s)

            return _

        lax.fori_loop(0, NG // RING, body, None)
        pltpu.sync_copy(rowbuf, out_hbm.at[pl.ds(base_row, ROWS_PER_W)])

    return gather_kernel(emb2, idx)


BLK = 2048
GRID = B // BLK


def _mlp_loss_body(x_ref, w1_ref, b1_ref, w2_ref, b2_ref, lab_ref, out_ref):
    x = x_ref[...]
    h = jnp.maximum(
        jnp.dot(x, w1_ref[...], preferred_element_type=jnp.float32)
        + b1_ref[...],
        0.0,
    )
    w2 = w2_ref[...]
    wd = w2[:, 0:1] - w2[:, 1:2]
    b2 = b2_ref[...]
    d = jnp.dot(h, wd, preferred_element_type=jnp.float32) + (
        b2[0, 0] - b2[0, 1]
    )
    sign = (2 * lab_ref[...] - 1).astype(jnp.float32)
    z = sign * d
    nll = jnp.maximum(z, 0.0) + jnp.log1p(jnp.exp(-jnp.abs(z)))
    partial = jnp.sum(nll) * (1.0 / B)

    @pl.when(pl.program_id(0) == 0)
    def _():
        out_ref[0, 0] = 0.0

    out_ref[0, 0] += partial


def _tc_mlp_loss(x, w1, b1, w2, b2, labels):
    return pl.pallas_call(
        _mlp_loss_body,
        grid=(GRID,),
        in_specs=[
            pl.BlockSpec((BLK, 2 * D), lambda i: (i, 0)),
            pl.BlockSpec((2 * D, D), lambda i: (0, 0)),
            pl.BlockSpec((1, D), lambda i: (0, 0)),
            pl.BlockSpec((D, 2), lambda i: (0, 0)),
            pl.BlockSpec((1, 2), lambda i: (0, 0)),
            pl.BlockSpec((BLK, 1), lambda i: (i, 0)),
        ],
        out_specs=pl.BlockSpec(
            (1, 1), lambda i: (0, 0), memory_space=pltpu.SMEM
        ),
        out_shape=jax.ShapeDtypeStruct((1, 1), jnp.float32),
    )(x, w1, b1, w2, b2, labels)


def kernel(pairs, labels, emb, W1, b1, W2, b2):
    idx = pairs.astype(jnp.int32).reshape(TOTAL)
    emb_rm = _tc_transpose(emb.T)
    x = _sc_gather(emb_rm, idx)
    loss = _tc_mlp_loss(
        x,
        W1,
        b1.reshape(1, D),
        W2,
        b2.reshape(1, 2),
        labels.astype(jnp.int32).reshape(B, 1),
    )
    return loss[0, 0]


# speedup vs baseline: 1.0564x; 1.0126x over previous
import functools

import jax
import jax.numpy as jnp
from jax import lax
from jax.experimental import pallas as pl
from jax.experimental.pallas import tpu as pltpu
from jax.experimental.pallas import tpu_sc as plsc

N = 1000000
D = 64
B = 16384

NC = 2
NS = 16
NW = NC * NS
TOTAL = 2 * B
IDX_PER_W = TOTAL // NW
ROWS_PER_W = IDX_PER_W // 2
PIPE = 64

TW = 32768
TW2 = TW // 2
TGRID = -(-N // TW)
LINES = TGRID * TW2


def _transpose_body(xt_ref, eye_ref, out_ref):
    xt = xt_ref[...]
    eye = eye_ref[...]
    dn = (((0,), (0,)), ((), ()))
    out_ref[:, 0:D] = lax.dot_general(
        xt[:, 0:TW2], eye, dn, preferred_element_type=jnp.float32
    )
    out_ref[:, D : 2 * D] = lax.dot_general(
        xt[:, TW2:TW], eye, dn, preferred_element_type=jnp.float32
    )


def _tc_transpose(embT):
    eye = jnp.eye(D, dtype=jnp.float32)
    return pl.pallas_call(
        _transpose_body,
        grid=(TGRID,),
        in_specs=[
            pl.BlockSpec((D, TW), lambda i: (0, i)),
            pl.BlockSpec((D, D), lambda i: (0, 0)),
        ],
        out_specs=pl.BlockSpec((TW2, 2 * D), lambda i: (i, 0)),
        out_shape=jax.ShapeDtypeStruct((LINES, 2 * D), jnp.float32),
    )(embT, eye)


GS = 16
NG = IDX_PER_W // GS
RING = 4


def _sc_gather(emb2, idx):
    mesh = plsc.VectorSubcoreMesh(core_axis_name="c", subcore_axis_name="s")

    @functools.partial(
        pl.kernel,
        out_type=jax.ShapeDtypeStruct((B, 2 * D), jnp.float32),
        mesh=mesh,
        scratch_types=[
            pltpu.VMEM((IDX_PER_W,), jnp.int32),
            pltpu.VMEM((RING, GS, 2 * D), jnp.float32),
            pltpu.VMEM((ROWS_PER_W, 2 * D), jnp.float32),
            [pltpu.SemaphoreType.DMA] * RING,
        ],
    )
    def gather_kernel(emb_hbm, idx_hbm, out_hbm, idx_v, stage, rowbuf, sems):
        wid = lax.axis_index("s") * NC + lax.axis_index("c")
        base_idx = wid * IDX_PER_W
        base_row = wid * ROWS_PER_W
        pltpu.sync_copy(idx_hbm.at[pl.ds(base_idx, IDX_PER_W)], idx_v)

        def issue(g, s):
            k16 = idx_v[pl.ds(g * GS, GS)]
            for j in range(GS):
                k = k16[j]
                line = lax.bitwise_or(
                    lax.shift_left(lax.shift_right_logical(k, 15), 14),
                    lax.bitwise_and(k, TW2 - 1),
                )
                pltpu.make_async_copy(
                    emb_hbm.at[line], stage.at[s, j], sems[s]
                ).start()

        def drain_extract(g, s):
            for j in range(GS):
                pltpu.make_async_copy(
                    emb_hbm.at[0], stage.at[s, 0], sems[s]
                ).wait()
            k16 = idx_v[pl.ds(g * GS, GS)]
            for j in range(GS):
                off = lax.bitwise_and(
                    lax.shift_right_logical(k16[j], 14), 1
                ) * D
                orow = g * (GS // 2) + (j // 2)
                colh = (j % 2) * D
                for t in range(D // 16):
                    rowbuf[orow, pl.ds(colh + t * 16, 16)] = stage[
                        s, j, pl.ds(off + t * 16, 16)
                    ]

        for s in range(RING):
            issue(s, s)

        def body(q, _):
            for s in range(RING):
                g = q * RING + s
                drain_extract(g, s)

                @pl.when(g + RING < NG)
                def _():
                    issue(g + RING, s)

            return _

        lax.fori_loop(0, NG // RING, body, None)
        pltpu.sync_copy(rowbuf, out_hbm.at[pl.ds(base_row, ROWS_PER_W)])

    return gather_kernel(emb2, idx)


BLK = 4096
GRID = B // BLK


def _mlp_loss_body(x_ref, w1_ref, b1_ref, w2_ref, b2_ref, lab_ref, out_ref):
    x = x_ref[...]
    h = jnp.maximum(
        jnp.dot(x, w1_ref[...], preferred_element_type=jnp.float32)
        + b1_ref[...],
        0.0,
    )
    w2 = w2_ref[...]
    wd = w2[:, 0:1] - w2[:, 1:2]
    b2 = b2_ref[...]
    d = jnp.dot(h, wd, preferred_element_type=jnp.float32) + (
        b2[0, 0] - b2[0, 1]
    )
    sign = (2 * lab_ref[...] - 1).astype(jnp.float32)
    z = sign * d
    nll = jnp.maximum(z, 0.0) + jnp.log1p(jnp.exp(-jnp.abs(z)))
    partial = jnp.sum(nll) * (1.0 / B)

    @pl.when(pl.program_id(0) == 0)
    def _():
        out_ref[0, 0] = 0.0

    out_ref[0, 0] += partial


def _tc_mlp_loss(x, w1, b1, w2, b2, labels):
    return pl.pallas_call(
        _mlp_loss_body,
        grid=(GRID,),
        in_specs=[
            pl.BlockSpec((BLK, 2 * D), lambda i: (i, 0)),
            pl.BlockSpec((2 * D, D), lambda i: (0, 0)),
            pl.BlockSpec((1, D), lambda i: (0, 0)),
            pl.BlockSpec((D, 2), lambda i: (0, 0)),
            pl.BlockSpec((1, 2), lambda i: (0, 0)),
            pl.BlockSpec((BLK, 1), lambda i: (i, 0)),
        ],
        out_specs=pl.BlockSpec(
            (1, 1), lambda i: (0, 0), memory_space=pltpu.SMEM
        ),
        out_shape=jax.ShapeDtypeStruct((1, 1), jnp.float32),
    )(x, w1, b1, w2, b2, labels)


def kernel(pairs, labels, emb, W1, b1, W2, b2):
    idx = pairs.astype(jnp.int32).reshape(TOTAL)
    emb_rm = _tc_transpose(emb.T)
    x = _sc_gather(emb_rm, idx)
    loss = _tc_mlp_loss(
        x,
        W1,
        b1.reshape(1, D),
        W2,
        b2.reshape(1, 2),
        labels.astype(jnp.int32).reshape(B, 1),
    )
    return loss[0, 0]


# speedup vs baseline: 1.0577x; 1.0012x over previous
import functools

import jax
import jax.numpy as jnp
from jax import lax
from jax.experimental import pallas as pl
from jax.experimental.pallas import tpu as pltpu
from jax.experimental.pallas import tpu_sc as plsc

N = 1000000
D = 64
B = 16384

NC = 2
NS = 16
NW = NC * NS
TOTAL = 2 * B
IDX_PER_W = TOTAL // NW
ROWS_PER_W = IDX_PER_W // 2
PIPE = 64

TW = 32768
TW2 = TW // 2
TGRID = -(-N // TW)
LINES = TGRID * TW2


def _transpose_body(xt_ref, eye_ref, out_ref):
    xt = xt_ref[...]
    eye = eye_ref[...]
    dn = (((0,), (0,)), ((), ()))
    out_ref[:, 0:D] = lax.dot_general(
        xt[:, 0:TW2], eye, dn, preferred_element_type=jnp.float32
    )
    out_ref[:, D : 2 * D] = lax.dot_general(
        xt[:, TW2:TW], eye, dn, preferred_element_type=jnp.float32
    )


def _tc_transpose(embT):
    eye = jnp.eye(D, dtype=jnp.float32)
    return pl.pallas_call(
        _transpose_body,
        grid=(TGRID,),
        in_specs=[
            pl.BlockSpec((D, TW), lambda i: (0, i)),
            pl.BlockSpec((D, D), lambda i: (0, 0)),
        ],
        out_specs=pl.BlockSpec((TW2, 2 * D), lambda i: (i, 0)),
        out_shape=jax.ShapeDtypeStruct((LINES, 2 * D), jnp.float32),
    )(embT, eye)


GS = 16
NG = IDX_PER_W // GS
RING = 4


def _sc_gather(emb2, idx):
    mesh = plsc.VectorSubcoreMesh(core_axis_name="c", subcore_axis_name="s")

    @functools.partial(
        pl.kernel,
        out_type=jax.ShapeDtypeStruct((B, 2 * D), jnp.float32),
        mesh=mesh,
        scratch_types=[
            pltpu.VMEM((IDX_PER_W,), jnp.int32),
            pltpu.VMEM((RING, GS, 2 * D), jnp.float32),
            pltpu.VMEM((ROWS_PER_W, 2 * D), jnp.float32),
            [pltpu.SemaphoreType.DMA] * RING,
        ],
    )
    def gather_kernel(emb_hbm, idx_hbm, out_hbm, idx_v, stage, rowbuf, sems):
        wid = lax.axis_index("s") * NC + lax.axis_index("c")
        base_idx = wid * IDX_PER_W
        base_row = wid * ROWS_PER_W
        pltpu.sync_copy(idx_hbm.at[pl.ds(base_idx, IDX_PER_W)], idx_v)

        def issue(g, s):
            k16 = idx_v[pl.ds(g * GS, GS)]
            for j in range(GS):
                k = k16[j]
                line = lax.bitwise_or(
                    lax.shift_left(lax.shift_right_logical(k, 15), 14),
                    lax.bitwise_and(k, TW2 - 1),
                )
                pltpu.make_async_copy(
                    emb_hbm.at[line], stage.at[s, j], sems[s]
                ).start()

        def drain_extract(g, s):
            for j in range(GS):
                pltpu.make_async_copy(
                    emb_hbm.at[0], stage.at[s, 0], sems[s]
                ).wait()
            k16 = idx_v[pl.ds(g * GS, GS)]
            for j in range(GS):
                off = lax.bitwise_and(
                    lax.shift_right_logical(k16[j], 14), 1
                ) * D
                orow = g * (GS // 2) + (j // 2)
                colh = (j % 2) * D
                for t in range(D // 16):
                    rowbuf[orow, pl.ds(colh + t * 16, 16)] = stage[
                        s, j, pl.ds(off + t * 16, 16)
                    ]

        for s in range(RING):
            issue(s, s)

        def body(q, _):
            for s in range(RING):
                g = q * RING + s
                drain_extract(g, s)

                @pl.when(g + RING < NG)
                def _():
                    issue(g + RING, s)

            return _

        lax.fori_loop(0, NG // RING, body, None)
        pltpu.sync_copy(rowbuf, out_hbm.at[pl.ds(base_row, ROWS_PER_W)])

    return gather_kernel(emb2, idx)


BLK = 8192
GRID = B // BLK


def _mlp_loss_body(x_ref, w1_ref, b1_ref, w2_ref, b2_ref, lab_ref, out_ref):
    x = x_ref[...]
    h = jnp.maximum(
        jnp.dot(x, w1_ref[...], preferred_element_type=jnp.float32)
        + b1_ref[...],
        0.0,
    )
    w2 = w2_ref[...]
    wd = w2[:, 0:1] - w2[:, 1:2]
    b2 = b2_ref[...]
    d = jnp.dot(h, wd, preferred_element_type=jnp.float32) + (
        b2[0, 0] - b2[0, 1]
    )
    sign = (2 * lab_ref[...] - 1).astype(jnp.float32)
    z = sign * d
    nll = jnp.maximum(z, 0.0) + jnp.log1p(jnp.exp(-jnp.abs(z)))
    partial = jnp.sum(nll) * (1.0 / B)

    @pl.when(pl.program_id(0) == 0)
    def _():
        out_ref[0, 0] = 0.0

    out_ref[0, 0] += partial


def _tc_mlp_loss(x, w1, b1, w2, b2, labels):
    return pl.pallas_call(
        _mlp_loss_body,
        grid=(GRID,),
        in_specs=[
            pl.BlockSpec((BLK, 2 * D), lambda i: (i, 0)),
            pl.BlockSpec((2 * D, D), lambda i: (0, 0)),
            pl.BlockSpec((1, D), lambda i: (0, 0)),
            pl.BlockSpec((D, 2), lambda i: (0, 0)),
            pl.BlockSpec((1, 2), lambda i: (0, 0)),
            pl.BlockSpec((BLK, 1), lambda i: (i, 0)),
        ],
        out_specs=pl.BlockSpec(
            (1, 1), lambda i: (0, 0), memory_space=pltpu.SMEM
        ),
        out_shape=jax.ShapeDtypeStruct((1, 1), jnp.float32),
    )(x, w1, b1, w2, b2, labels)


def kernel(pairs, labels, emb, W1, b1, W2, b2):
    idx = pairs.astype(jnp.int32).reshape(TOTAL)
    emb_rm = _tc_transpose(emb.T)
    x = _sc_gather(emb_rm, idx)
    loss = _tc_mlp_loss(
        x,
        W1,
        b1.reshape(1, D),
        W2,
        b2.reshape(1, 2),
        labels.astype(jnp.int32).reshape(B, 1),
    )
    return loss[0, 0]
